# baseline (speedup 1.0000x reference)
import functools
import math

import jax
import jax.numpy as jnp
from jax import lax
from jax.experimental import pallas as pl
from jax.experimental.pallas import tpu as pltpu
from jax.experimental.pallas import tpu_sc as plsc

VOCAB = 100000
D = 768
B = 4
S = 4096
N = B * S
SCALE = math.sqrt(float(D))

_info = plsc.get_sparse_core_info()
NC = _info.num_cores
NS = _info.num_subcores
NW = NC * NS
S_W = S // NW
R = 32
NCH = S_W // R
LANES = 16
JV = D // LANES


def _sc_embed(idx_arr, table, pe_s):
    mesh = plsc.VectorSubcoreMesh(core_axis_name="c", subcore_axis_name="s")

    @functools.partial(
        pl.kernel,
        mesh=mesh,
        out_type=jax.ShapeDtypeStruct((N, D), jnp.float32),
        scratch_types=[
            pltpu.VMEM((NCH * B, R), jnp.int32),
            pltpu.VMEM((B, R, D), jnp.float32),
            pltpu.VMEM((R, D), jnp.float32),
            pltpu.SemaphoreType.DMA,
            pltpu.SemaphoreType.DMA((B,)),
        ],
    )
    def k(idx_hbm, table_hbm, pe_hbm, out_hbm,
          idx_v, rows_v, pe_v, g_sem, o_sem):
        wid = lax.axis_index("s") * NC + lax.axis_index("c")
        sbase = wid * S_W

        def drain_out(b):
            pltpu.make_async_copy(
                rows_v.at[b], out_hbm.at[pl.ds(0, R)], o_sem.at[b]).wait()

        pltpu.sync_copy(idx_hbm.at[wid], idx_v)

        def chunk(sc, _):
            pltpu.sync_copy(pe_hbm.at[pl.ds(sbase + sc * R, R)], pe_v)
            for b in range(B):
                pl.when(sc >= 1)(lambda b=b: drain_out(b))
                pltpu.async_copy(
                    table_hbm.at[idx_v.at[sc * B + b]],
                    rows_v.at[b], g_sem).wait()

            def row(r, _):
                for j in range(JV):
                    sl = pl.ds(j * LANES, LANES)
                    pe_reg = pe_v[r, sl]
                    for b in range(B):
                        rows_v[b, r, sl] = rows_v[b, r, sl] * SCALE + pe_reg
                return 0

            lax.fori_loop(0, R, row, 0)
            for b in range(B):
                pltpu.async_copy(
                    rows_v.at[b],
                    out_hbm.at[pl.ds(b * S + sbase + sc * R, R)],
                    o_sem.at[b])
            return 0

        lax.fori_loop(0, NCH, chunk, 0)
        for b in range(B):
            drain_out(b)

    return k(idx_arr, table, pe_s)


def kernel(x, table, pe):
    idx_arr = (x.reshape(B, NW, NCH, R)
                .transpose(1, 2, 0, 3)
                .reshape(NW, NCH * B, R))
    out = _sc_embed(idx_arr, table, pe[:S])
    return out.reshape(B, S, D)


# speedup vs baseline: 1.0810x; 1.0810x over previous
import functools
import math

import jax
import jax.numpy as jnp
from jax import lax
from jax.experimental import pallas as pl
from jax.experimental.pallas import tpu as pltpu
from jax.experimental.pallas import tpu_sc as plsc

VOCAB = 100000
D = 768
B = 4
S = 4096
N = B * S
SCALE = math.sqrt(float(D))

_info = plsc.get_sparse_core_info()
NC = _info.num_cores
NS = _info.num_subcores
NW = NC * NS
S_W = S // NW
R = 32
NCH = S_W // R
LANES = 16
JV = D // LANES


def _sc_embed(idx_arr, table, pe_s):
    mesh = plsc.VectorSubcoreMesh(core_axis_name="c", subcore_axis_name="s")

    @functools.partial(
        pl.kernel,
        mesh=mesh,
        out_type=jax.ShapeDtypeStruct((N, D), jnp.float32),
        scratch_types=[
            pltpu.VMEM((NCH * B, R), jnp.int32),
            pltpu.VMEM((2, R, D), jnp.float32),
            pltpu.VMEM((2, R, D), jnp.float32),
            pltpu.SemaphoreType.DMA,
            pltpu.SemaphoreType.DMA((2,)),
            pltpu.SemaphoreType.DMA((2,)),
        ],
    )
    def k(idx_hbm, table_hbm, pe_hbm, out_hbm,
          idx_v, rows_v, pe_v, g_sem, o_sem, p_sem):
        wid = lax.axis_index("s") * NC + lax.axis_index("c")
        sbase = wid * S_W

        def drain_out(par):
            pltpu.make_async_copy(
                rows_v.at[par], out_hbm.at[pl.ds(0, R)], o_sem.at[par]).wait()

        def fire_pe(sc, par):
            pltpu.async_copy(
                pe_hbm.at[pl.ds(sbase + sc * R, R)], pe_v.at[par],
                p_sem.at[par])

        def drain_pe(sc, par):
            pltpu.make_async_copy(
                pe_hbm.at[pl.ds(sbase + sc * R, R)], pe_v.at[par],
                p_sem.at[par]).wait()

        pltpu.sync_copy(idx_hbm.at[wid], idx_v)
        fire_pe(0, 0)

        def stage(g, off):
            sc = 2 * g + off
            pe_par = off
            drain_pe(sc, pe_par)

            def pf():
                fire_pe(sc + 1, 1 - pe_par)
            if off == 0:
                pf()
            else:
                pl.when(g < NCH // 2 - 1)(pf)
            for b in range(B):
                par = b % 2
                if off == 0 and b < 2:
                    pl.when(g >= 1)(lambda par=par: drain_out(par))
                else:
                    drain_out(par)
                pltpu.async_copy(
                    table_hbm.at[idx_v.at[sc * B + b]],
                    rows_v.at[par], g_sem).wait()

                def row(r, _, par=par, pe_par=pe_par):
                    for j in range(JV):
                        sl = pl.ds(j * LANES, LANES)
                        rows_v[par, r, sl] = (
                            rows_v[par, r, sl] * SCALE + pe_v[pe_par, r, sl])
                    return 0

                lax.fori_loop(0, R, row, 0)
                pltpu.async_copy(
                    rows_v.at[par],
                    out_hbm.at[pl.ds(b * S + sbase + sc * R, R)],
                    o_sem.at[par])

        def group(g, _):
            stage(g, 0)
            stage(g, 1)
            return 0

        lax.fori_loop(0, NCH // 2, group, 0)
        drain_out(0)
        drain_out(1)

    return k(idx_arr, table, pe_s)


def kernel(x, table, pe):
    idx_arr = (x.reshape(B, NW, NCH, R)
                .transpose(1, 2, 0, 3)
                .reshape(NW, NCH * B, R))
    out = _sc_embed(idx_arr, table, pe[:S])
    return out.reshape(B, S, D)


# speedup vs baseline: 1.4563x; 1.3472x over previous
import functools
import math

import jax
import jax.numpy as jnp
from jax import lax
from jax.experimental import pallas as pl
from jax.experimental.pallas import tpu as pltpu
from jax.experimental.pallas import tpu_sc as plsc

VOCAB = 100000
D = 768
B = 4
S = 4096
N = B * S
SCALE = math.sqrt(float(D))

_info = plsc.get_sparse_core_info()
NC = _info.num_cores
NS = _info.num_subcores
NW = NC * NS
S_W = S // NW
R = 32
NCH = S_W // R
LANES = 16
JV = D // LANES


def _sc_embed(x_flat, table, pe_s):
    mesh = plsc.VectorSubcoreMesh(core_axis_name="c", subcore_axis_name="s")

    @functools.partial(
        pl.kernel,
        mesh=mesh,
        out_type=jax.ShapeDtypeStruct((N, D), jnp.float32),
        scratch_types=[
            pltpu.VMEM((B * S_W,), jnp.int32),
            pltpu.VMEM((2, R, D), jnp.float32),
            pltpu.VMEM((R, D), jnp.float32),
            pltpu.SemaphoreType.DMA,
            pltpu.SemaphoreType.DMA,
            pltpu.SemaphoreType.DMA,
        ],
    )
    def k(idx_hbm, table_hbm, pe_hbm, out_hbm,
          idx_v, rows_v, pe_v, g_sem, o0, o1):
        wid = lax.axis_index("s") * NC + lax.axis_index("c")
        sbase = wid * S_W
        o_sem = (o0, o1)

        def drain_out(par):
            pltpu.make_async_copy(
                rows_v.at[par], out_hbm.at[pl.ds(0, R)], o_sem[par]).wait()

        for b in range(B):
            pltpu.sync_copy(
                idx_hbm.at[pl.ds(b * S + sbase, S_W)],
                idx_v.at[pl.ds(b * S_W, S_W)])

        def chunk(sc, _):
            pltpu.sync_copy(pe_hbm.at[pl.ds(sbase + sc * R, R)], pe_v)
            for b in range(B):
                par = b % 2
                if b < 2:
                    pl.when(sc >= 1)(lambda par=par: drain_out(par))
                else:
                    drain_out(par)
                pltpu.async_copy(
                    table_hbm.at[idx_v.at[pl.ds(b * S_W + sc * R, R)]],
                    rows_v.at[par], g_sem).wait()

                def row(r, _, par=par):
                    for j in range(JV):
                        sl = pl.ds(j * LANES, LANES)
                        rows_v[par, r, sl] = (
                            rows_v[par, r, sl] * SCALE + pe_v[r, sl])
                    return 0

                lax.fori_loop(0, R, row, 0)
                pltpu.async_copy(
                    rows_v.at[par],
                    out_hbm.at[pl.ds(b * S + sbase + sc * R, R)], o_sem[par])
            return 0

        lax.fori_loop(0, NCH, chunk, 0)
        drain_out(0)
        drain_out(1)

    return k(x_flat, table, pe_s)


def kernel(x, table, pe):
    out = _sc_embed(x.reshape(N), table, pe[:S])
    return out.reshape(B, S, D)
